# baseline (speedup 1.0000x reference)
import functools

import numpy as np
import jax
import jax.numpy as jnp
from jax import lax
from jax.experimental import pallas as pl
from jax.experimental.pallas import tpu as pltpu
from jax.experimental.pallas import tpu_sc as plsc

_NUM_CLASSES = 80
_CH = 5 + _NUM_CLASSES
_N = 22743
_NPAD = 24576
_NSUB = 16
_SH = _NPAD // _NSUB
_NCHUNK = _SH // 16
_MAXB = 100
_IOU_T = 0.5
_SCORE_T = 0.1
_NEG = -1e30
_GUARD = -1e20
_GRIDS = (76, 38, 19)
_TC_BLK = 4096


def _grid_consts():
    cgx, cgy, gf = [], [], []
    for g in _GRIDS:
        xs, ys = np.meshgrid(np.arange(g), np.arange(g))
        cgx.append(np.repeat(xs.reshape(-1), 3))
        cgy.append(np.repeat(ys.reshape(-1), 3))
        gf.append(np.full(3 * g * g, g, np.float32))
    pad = _NPAD - _N
    cgx = np.concatenate(cgx + [np.zeros(pad)]).astype(np.float32)
    cgy = np.concatenate(cgy + [np.zeros(pad)]).astype(np.float32)
    gf = np.concatenate(gf + [np.ones(pad, np.float32)]).astype(np.float32)
    return cgx[None], cgy[None], gf[None]


_CGX, _CGY, _GF = _grid_consts()


def _decode_body(p_ref, cgx_ref, cgy_ref, gf_ref, aw_ref, ah_ref, *out_ref):
    p = p_ref[...]

    def sig(x):
        return 1.0 / (1.0 + jnp.exp(-x))

    gf = gf_ref[...]
    bx = (sig(p[0:1]) + cgx_ref[...]) / gf
    by = (sig(p[1:2]) + cgy_ref[...]) / gf
    w = jnp.exp(p[2:3]) * aw_ref[...]
    h = jnp.exp(p[3:4]) * ah_ref[...]
    x1 = bx - w / 2.0
    y1 = by - h / 2.0
    x2 = bx + w / 2.0
    y2 = by + h / 2.0
    area = (x2 - x1) * (y2 - y1)
    cl = p[5:_CH]
    m = jnp.max(cl, axis=0, keepdims=True)
    iot = lax.broadcasted_iota(jnp.int32, cl.shape, 0)
    am = jnp.min(jnp.where(cl == m, iot, 1000000), axis=0, keepdims=True)
    score = sig(p[4:5]) * sig(m)
    for ref, val in zip(out_ref, (x1, y1, x2, y2, area, score,
                                  am.astype(jnp.float32))):
        ref[...] = val[0]


_decode = pl.pallas_call(
    _decode_body,
    grid=(_NPAD // _TC_BLK,),
    in_specs=[pl.BlockSpec((_CH, _TC_BLK), lambda i: (0, i))]
    + [pl.BlockSpec((1, _TC_BLK), lambda i: (0, i))] * 5,
    out_specs=[pl.BlockSpec((_TC_BLK,), lambda i: (i,))] * 7,
    out_shape=[jax.ShapeDtypeStruct((_NPAD,), jnp.float32)] * 7,
)


def _nms_body(x1h, y1h, x2h, y2h, arh, sch, clh, out_hbm, x1v, y1v, x2v, y2v,
              arv, clv, effv, candv, tmpv, obufv, cshr):
    wid = lax.axis_index("s")
    base = wid * _SH
    iota = lax.iota(jnp.int32, 16)
    iotaf = iota.astype(jnp.float32)
    basef = (base * jnp.float32(1.0)).astype(jnp.float32)

    pltpu.sync_copy(x1h.at[pl.ds(base, _SH)], x1v)
    pltpu.sync_copy(y1h.at[pl.ds(base, _SH)], y1v)
    pltpu.sync_copy(x2h.at[pl.ds(base, _SH)], x2v)
    pltpu.sync_copy(y2h.at[pl.ds(base, _SH)], y2v)
    pltpu.sync_copy(arh.at[pl.ds(base, _SH)], arv)
    pltpu.sync_copy(sch.at[pl.ds(base, _SH)], effv)
    pltpu.sync_copy(clh.at[pl.ds(base, _SH)], clv)

    neg16 = jnp.full((16,), _NEG, jnp.float32)
    zero16 = jnp.zeros((16,), jnp.float32)

    def init_chunk(j, c):
        bv, bi = c
        sl = pl.ds(j * 16, 16)
        gidx = basef + (j * 16) * jnp.float32(1.0) + iotaf
        v = jnp.where(gidx >= jnp.float32(_N), jnp.float32(-1.0), effv[sl])
        effv[sl] = v
        upd = v > bv
        return jnp.where(upd, v, bv), jnp.where(upd, gidx, bi)

    bv0, bi0 = lax.fori_loop(0, _NCHUNK, init_chunk, (neg16, zero16))

    zi16 = jnp.zeros((16,), jnp.int32)
    b0x1 = plsc.load_gather(x1v, [zi16])
    b0y1 = plsc.load_gather(y1v, [zi16])
    b0x2 = plsc.load_gather(x2v, [zi16])
    b0y2 = plsc.load_gather(y2v, [zi16])
    b0cl = plsc.load_gather(clv, [zi16])

    def itbody(i, carry):
        bv, bi, nv = carry
        lm = jnp.max(bv)
        li = -jnp.max(jnp.where(bv == lm, -bi, jnp.float32(_NEG)))
        loff = (jnp.full((16,), li) - basef).astype(jnp.int32)
        row = jnp.where(iota == 0, jnp.full((16,), lm), zero16)
        row = jnp.where(iota == 1, jnp.full((16,), li), row)
        row = jnp.where(iota == 2, plsc.load_gather(x1v, [loff]), row)
        row = jnp.where(iota == 3, plsc.load_gather(y1v, [loff]), row)
        row = jnp.where(iota == 4, plsc.load_gather(x2v, [loff]), row)
        row = jnp.where(iota == 5, plsc.load_gather(y2v, [loff]), row)
        row = jnp.where(iota == 6, plsc.load_gather(arv, [loff]), row)
        row = jnp.where(iota == 7, plsc.load_gather(clv, [loff]), row)
        tmpv[...] = row
        pltpu.sync_copy(tmpv, cshr.at[wid, pl.ds(0, 16)])
        plsc.subcore_barrier()
        pltpu.sync_copy(cshr, candv)
        plsc.subcore_barrier()

        zi = jnp.zeros((16,), jnp.int32)
        s = plsc.load_gather(candv, [iota, zi])
        g = jnp.max(s)
        ownf = -jnp.max(jnp.where(s == g, -iotaf, jnp.float32(_NEG)))
        owni = jnp.full((16,), ownf).astype(jnp.int32)

        def fld(k):
            return plsc.load_gather(candv, [owni, jnp.full((16,), k, jnp.int32)])

        widxv = fld(1)
        wx1 = fld(2)
        wy1 = fld(3)
        wx2 = fld(4)
        wy2 = fld(5)
        wav = fld(6)
        wclv = fld(7)
        valid = g > jnp.float32(_SCORE_T)
        validv = jnp.full((16,), g) > jnp.float32(_SCORE_T)
        nv = nv + jnp.where(valid, jnp.float32(1.0), jnp.float32(0.0))

        @pl.when(wid == 0)
        def _():
            orow = jnp.where(iota == 0, jnp.where(validv, wx1, b0x1), zero16)
            orow = jnp.where(iota == 1, jnp.where(validv, wy1, b0y1), orow)
            orow = jnp.where(iota == 2, jnp.where(validv, wx2, b0x2), orow)
            orow = jnp.where(iota == 3, jnp.where(validv, wy2, b0y2), orow)
            orow = jnp.where(iota == 4,
                             jnp.where(validv, jnp.full((16,), g), zero16), orow)
            orow = jnp.where(iota == 5, jnp.where(validv, wclv, b0cl), orow)
            plsc.store_scatter(obufv, [jnp.full((16,), i, jnp.int32), iota], orow)

        def rw(j, c2):
            bv2, bi2 = c2
            sl = pl.ds(j * 16, 16)
            ix1 = jnp.maximum(wx1, x1v[sl])
            iy1 = jnp.maximum(wy1, y1v[sl])
            ix2 = jnp.minimum(wx2, x2v[sl])
            iy2 = jnp.minimum(wy2, y2v[sl])
            inter = jnp.maximum(ix2 - ix1, 0.0) * jnp.maximum(iy2 - iy1, 0.0)
            iou = inter / (wav + arv[sl] - inter + jnp.float32(1e-9))
            wgt = jnp.where(iou <= jnp.float32(_IOU_T),
                            jnp.exp(-iou * iou), jnp.float32(0.0))
            wgt = jnp.where(validv, wgt, jnp.float32(1.0))
            ce = effv[sl]
            ne = jnp.where(ce > jnp.float32(_GUARD), ce * wgt, ce)
            gidx = basef + (j * 16) * jnp.float32(1.0) + iotaf
            ne = jnp.where(validv & (gidx == widxv), jnp.float32(_NEG), ne)
            effv[sl] = ne
            upd = ne > bv2
            return jnp.where(upd, ne, bv2), jnp.where(upd, gidx, bi2)

        bv, bi = lax.fori_loop(0, _NCHUNK, rw, (neg16, zero16))
        return bv, bi, nv

    bv, bi, nv = lax.fori_loop(0, _MAXB, itbody, (bv0, bi0, jnp.float32(0.0)))

    @pl.when(wid == 0)
    def _():
        plsc.store_scatter(obufv, [jnp.full((16,), _MAXB, jnp.int32), iota],
                           jnp.full((16,), nv))
        pltpu.sync_copy(obufv, out_hbm)


@functools.cache
def _nms_call():
  return functools.partial(
    pl.kernel,
    out_type=jax.ShapeDtypeStruct((128, 16), jnp.float32),
    mesh=plsc.VectorSubcoreMesh(core_axis_name="c", subcore_axis_name="s",
                                num_cores=1, num_subcores=_NSUB),
    compiler_params=pltpu.CompilerParams(needs_layout_passes=False),
    scratch_types=[
        pltpu.VMEM((_SH,), jnp.float32),
        pltpu.VMEM((_SH,), jnp.float32),
        pltpu.VMEM((_SH,), jnp.float32),
        pltpu.VMEM((_SH,), jnp.float32),
        pltpu.VMEM((_SH,), jnp.float32),
        pltpu.VMEM((_SH,), jnp.float32),
        pltpu.VMEM((_SH,), jnp.float32),
        pltpu.VMEM((16, 128), jnp.float32),
        pltpu.VMEM((16,), jnp.float32),
        pltpu.VMEM((128, 16), jnp.float32),
        pltpu.VMEM_SHARED((16, 128), jnp.float32),
    ],
  )(_nms_body)


def kernel(pred0, pred1, pred2, anchors0, anchors1, anchors2):
    p = jnp.concatenate([
        pred0.reshape(-1, _CH),
        pred1.reshape(-1, _CH),
        pred2.reshape(-1, _CH),
    ], axis=0)
    pT = jnp.pad(p.T, ((0, 0), (0, _NPAD - _N)))

    def expand(anc, g):
        return jnp.tile(anc, (g * g, 1))

    anc = jnp.concatenate([expand(anchors0, 76), expand(anchors1, 38),
                           expand(anchors2, 19)], axis=0)
    aw = jnp.pad(anc[:, 0], (0, _NPAD - _N))[None]
    ah = jnp.pad(anc[:, 1], (0, _NPAD - _N))[None]

    dec = _decode(pT, jnp.asarray(_CGX), jnp.asarray(_CGY), jnp.asarray(_GF),
                  aw, ah)
    out = _nms_call()(*dec)
    boxes = out[:_MAXB, 0:4][None]
    scores = out[:_MAXB, 4][None]
    classes = out[:_MAXB, 5].astype(jnp.int32)[None]
    valid = out[_MAXB, 0].astype(jnp.int32)[None]
    return boxes, scores, classes, valid


# speedup vs baseline: 1.0576x; 1.0576x over previous
import functools

import numpy as np
import jax
import jax.numpy as jnp
from jax import lax
from jax.experimental import pallas as pl
from jax.experimental.pallas import tpu as pltpu
from jax.experimental.pallas import tpu_sc as plsc

_NUM_CLASSES = 80
_CH = 5 + _NUM_CLASSES
_N = 22743
_NPAD = 23040
_NSUB = 16
_SH = _NPAD // _NSUB
_NCHUNK = _SH // 16
_MAXB = 100
_IOU_T = 0.5
_SCORE_T = 0.1
_NEG = -1e30
_GUARD = -1e20
_GRIDS = (76, 38, 19)
_TC_BLK = _NPAD


def _grid_consts():
    cgx, cgy, gf = [], [], []
    for g in _GRIDS:
        xs, ys = np.meshgrid(np.arange(g), np.arange(g))
        cgx.append(np.repeat(xs.reshape(-1), 3))
        cgy.append(np.repeat(ys.reshape(-1), 3))
        gf.append(np.full(3 * g * g, g, np.float32))
    pad = _NPAD - _N
    cgx = np.concatenate(cgx + [np.zeros(pad)]).astype(np.float32)
    cgy = np.concatenate(cgy + [np.zeros(pad)]).astype(np.float32)
    gf = np.concatenate(gf + [np.ones(pad, np.float32)]).astype(np.float32)
    return cgx[None], cgy[None], gf[None]


_CGX, _CGY, _GF = _grid_consts()


def _decode_body(p_ref, cgx_ref, cgy_ref, gf_ref, aw_ref, ah_ref, *out_ref):
    p = p_ref[...]

    def sig(x):
        return 1.0 / (1.0 + jnp.exp(-x))

    gf = gf_ref[...]
    bx = (sig(p[0:1]) + cgx_ref[...]) / gf
    by = (sig(p[1:2]) + cgy_ref[...]) / gf
    w = jnp.exp(p[2:3]) * aw_ref[...]
    h = jnp.exp(p[3:4]) * ah_ref[...]
    x1 = bx - w / 2.0
    y1 = by - h / 2.0
    x2 = bx + w / 2.0
    y2 = by + h / 2.0
    area = (x2 - x1) * (y2 - y1)
    cl = p[5:_CH]
    m = jnp.max(cl, axis=0, keepdims=True)
    iot = lax.broadcasted_iota(jnp.int32, cl.shape, 0)
    am = jnp.min(jnp.where(cl == m, iot, 1000000), axis=0, keepdims=True)
    score = sig(p[4:5]) * sig(m)
    for ref, val in zip(out_ref, (x1, y1, x2, y2, area, score,
                                  am.astype(jnp.float32))):
        ref[...] = val[0]


_decode = pl.pallas_call(
    _decode_body,
    out_shape=[jax.ShapeDtypeStruct((_NPAD,), jnp.float32)] * 7,
)


def _nms_body(x1h, y1h, x2h, y2h, arh, sch, clh, out_hbm, x1v, y1v, x2v, y2v,
              arv, clv, effv, candv, tmpv, obufv, cshr):
    wid = lax.axis_index("s")
    base = wid * _SH
    iota = lax.iota(jnp.int32, 16)
    iotaf = iota.astype(jnp.float32)
    basef = (base * jnp.float32(1.0)).astype(jnp.float32)

    pltpu.sync_copy(x1h.at[pl.ds(base, _SH)], x1v)
    pltpu.sync_copy(y1h.at[pl.ds(base, _SH)], y1v)
    pltpu.sync_copy(x2h.at[pl.ds(base, _SH)], x2v)
    pltpu.sync_copy(y2h.at[pl.ds(base, _SH)], y2v)
    pltpu.sync_copy(arh.at[pl.ds(base, _SH)], arv)
    pltpu.sync_copy(sch.at[pl.ds(base, _SH)], effv)
    pltpu.sync_copy(clh.at[pl.ds(base, _SH)], clv)

    neg16 = jnp.full((16,), _NEG, jnp.float32)
    zero16 = jnp.zeros((16,), jnp.float32)

    def init_chunk(j, c):
        bv, bi = c
        sl = pl.ds(j * 16, 16)
        gidx = basef + (j * 16) * jnp.float32(1.0) + iotaf
        v = jnp.where(gidx >= jnp.float32(_N), jnp.float32(-1.0), effv[sl])
        effv[sl] = v
        upd = v > bv
        return jnp.where(upd, v, bv), jnp.where(upd, gidx, bi)

    bv0, bi0 = lax.fori_loop(0, _NCHUNK, init_chunk, (neg16, zero16))

    zi16 = jnp.zeros((16,), jnp.int32)
    b0x1 = plsc.load_gather(x1v, [zi16])
    b0y1 = plsc.load_gather(y1v, [zi16])
    b0x2 = plsc.load_gather(x2v, [zi16])
    b0y2 = plsc.load_gather(y2v, [zi16])
    b0cl = plsc.load_gather(clv, [zi16])

    def itbody(i, carry):
        bv, bi, nv = carry
        lm = jnp.max(bv)
        li = -jnp.max(jnp.where(bv == lm, -bi, jnp.float32(_NEG)))
        loff = (jnp.full((16,), li) - basef).astype(jnp.int32)
        row = jnp.where(iota == 0, jnp.full((16,), lm), zero16)
        row = jnp.where(iota == 1, jnp.full((16,), li), row)
        row = jnp.where(iota == 2, plsc.load_gather(x1v, [loff]), row)
        row = jnp.where(iota == 3, plsc.load_gather(y1v, [loff]), row)
        row = jnp.where(iota == 4, plsc.load_gather(x2v, [loff]), row)
        row = jnp.where(iota == 5, plsc.load_gather(y2v, [loff]), row)
        row = jnp.where(iota == 6, plsc.load_gather(arv, [loff]), row)
        row = jnp.where(iota == 7, plsc.load_gather(clv, [loff]), row)
        tmpv[...] = row
        pltpu.sync_copy(tmpv, cshr.at[wid, pl.ds(0, 16)])
        plsc.subcore_barrier()
        pltpu.sync_copy(cshr, candv)
        plsc.subcore_barrier()

        zi = jnp.zeros((16,), jnp.int32)
        s = plsc.load_gather(candv, [iota, zi])
        g = jnp.max(s)
        ownf = -jnp.max(jnp.where(s == g, -iotaf, jnp.float32(_NEG)))
        owni = jnp.full((16,), ownf).astype(jnp.int32)

        def fld(k):
            return plsc.load_gather(candv, [owni, jnp.full((16,), k, jnp.int32)])

        widxv = fld(1)
        wx1 = fld(2)
        wy1 = fld(3)
        wx2 = fld(4)
        wy2 = fld(5)
        wav = fld(6)
        wclv = fld(7)
        valid = g > jnp.float32(_SCORE_T)
        validv = jnp.full((16,), g) > jnp.float32(_SCORE_T)
        nv = nv + jnp.where(valid, jnp.float32(1.0), jnp.float32(0.0))

        @pl.when(wid == 0)
        def _():
            orow = jnp.where(iota == 0, jnp.where(validv, wx1, b0x1), zero16)
            orow = jnp.where(iota == 1, jnp.where(validv, wy1, b0y1), orow)
            orow = jnp.where(iota == 2, jnp.where(validv, wx2, b0x2), orow)
            orow = jnp.where(iota == 3, jnp.where(validv, wy2, b0y2), orow)
            orow = jnp.where(iota == 4,
                             jnp.where(validv, jnp.full((16,), g), zero16), orow)
            orow = jnp.where(iota == 5, jnp.where(validv, wclv, b0cl), orow)
            plsc.store_scatter(obufv, [jnp.full((16,), i, jnp.int32), iota], orow)

        owns = valid & (ownf == (wid * jnp.float32(1.0)))

        @pl.when(owns)
        def _():
            loffw = (widxv - basef).astype(jnp.int32)
            plsc.store_scatter(effv, [loffw], neg16, mask=iota == 0)

        def rw(j, c2):
            bv2, bi2 = c2
            sl = pl.ds(j * 16, 16)
            ix1 = jnp.maximum(wx1, x1v[sl])
            iy1 = jnp.maximum(wy1, y1v[sl])
            ix2 = jnp.minimum(wx2, x2v[sl])
            iy2 = jnp.minimum(wy2, y2v[sl])
            inter = jnp.maximum(ix2 - ix1, 0.0) * jnp.maximum(iy2 - iy1, 0.0)
            iou = inter / (wav + arv[sl] - inter + jnp.float32(1e-9))
            wgt = jnp.where(iou <= jnp.float32(_IOU_T),
                            jnp.exp(-iou * iou), jnp.float32(0.0))
            wgt = jnp.where(validv, wgt, jnp.float32(1.0))
            ce = effv[sl]
            ne = jnp.where(ce > jnp.float32(_GUARD), ce * wgt, ce)
            effv[sl] = ne
            gidx = basef + (j * 16) * jnp.float32(1.0) + iotaf
            upd = ne > bv2
            return jnp.where(upd, ne, bv2), jnp.where(upd, gidx, bi2)

        bv, bi = lax.fori_loop(0, _NCHUNK, rw, (neg16, zero16))
        return bv, bi, nv

    bv, bi, nv = lax.fori_loop(0, _MAXB, itbody, (bv0, bi0, jnp.float32(0.0)))

    @pl.when(wid == 0)
    def _():
        plsc.store_scatter(obufv, [jnp.full((16,), _MAXB, jnp.int32), iota],
                           jnp.full((16,), nv))
        pltpu.sync_copy(obufv, out_hbm)


@functools.cache
def _nms_call():
  return functools.partial(
    pl.kernel,
    out_type=jax.ShapeDtypeStruct((128, 16), jnp.float32),
    mesh=plsc.VectorSubcoreMesh(core_axis_name="c", subcore_axis_name="s",
                                num_cores=1, num_subcores=_NSUB),
    compiler_params=pltpu.CompilerParams(needs_layout_passes=False),
    scratch_types=[
        pltpu.VMEM((_SH,), jnp.float32),
        pltpu.VMEM((_SH,), jnp.float32),
        pltpu.VMEM((_SH,), jnp.float32),
        pltpu.VMEM((_SH,), jnp.float32),
        pltpu.VMEM((_SH,), jnp.float32),
        pltpu.VMEM((_SH,), jnp.float32),
        pltpu.VMEM((_SH,), jnp.float32),
        pltpu.VMEM((16, 128), jnp.float32),
        pltpu.VMEM((16,), jnp.float32),
        pltpu.VMEM((128, 16), jnp.float32),
        pltpu.VMEM_SHARED((16, 128), jnp.float32),
    ],
  )(_nms_body)


def kernel(pred0, pred1, pred2, anchors0, anchors1, anchors2):
    p = jnp.concatenate([
        pred0.reshape(-1, _CH),
        pred1.reshape(-1, _CH),
        pred2.reshape(-1, _CH),
    ], axis=0)
    pT = jnp.pad(p.T, ((0, 0), (0, _NPAD - _N)))

    def expand(anc, g):
        return jnp.tile(anc, (g * g, 1))

    anc = jnp.concatenate([expand(anchors0, 76), expand(anchors1, 38),
                           expand(anchors2, 19)], axis=0)
    aw = jnp.pad(anc[:, 0], (0, _NPAD - _N))[None]
    ah = jnp.pad(anc[:, 1], (0, _NPAD - _N))[None]

    dec = _decode(pT, jnp.asarray(_CGX), jnp.asarray(_CGY), jnp.asarray(_GF),
                  aw, ah)
    out = _nms_call()(*dec)
    boxes = out[:_MAXB, 0:4][None]
    scores = out[:_MAXB, 4][None]
    classes = out[:_MAXB, 5].astype(jnp.int32)[None]
    valid = out[_MAXB, 0].astype(jnp.int32)[None]
    return boxes, scores, classes, valid


# speedup vs baseline: 1.1047x; 1.0445x over previous
import functools

import numpy as np
import jax
import jax.numpy as jnp
from jax import lax
from jax.experimental import pallas as pl
from jax.experimental.pallas import tpu as pltpu
from jax.experimental.pallas import tpu_sc as plsc

_NUM_CLASSES = 80
_CH = 5 + _NUM_CLASSES
_N = 22743
_NPAD = 23040
_NSUB = 16
_SH = _NPAD // _NSUB
_NCHUNK = _SH // 16
_MAXB = 100
_IOU_T = 0.5
_SCORE_T = 0.1
_NEG = -1e30
_GUARD = -1e20
_GRIDS = (76, 38, 19)
_TC_BLK = _NPAD


def _grid_consts():
    cgx, cgy, gf = [], [], []
    for g in _GRIDS:
        xs, ys = np.meshgrid(np.arange(g), np.arange(g))
        cgx.append(np.repeat(xs.reshape(-1), 3))
        cgy.append(np.repeat(ys.reshape(-1), 3))
        gf.append(np.full(3 * g * g, g, np.float32))
    pad = _NPAD - _N
    cgx = np.concatenate(cgx + [np.zeros(pad)]).astype(np.float32)
    cgy = np.concatenate(cgy + [np.zeros(pad)]).astype(np.float32)
    gf = np.concatenate(gf + [np.ones(pad, np.float32)]).astype(np.float32)
    return cgx[None], cgy[None], gf[None]


_CGX, _CGY, _GF = _grid_consts()


def _decode_body(p_ref, cgx_ref, cgy_ref, gf_ref, aw_ref, ah_ref, *out_ref):
    p = p_ref[...]

    def sig(x):
        return 1.0 / (1.0 + jnp.exp(-x))

    gf = gf_ref[...]
    bx = (sig(p[0:1]) + cgx_ref[...]) / gf
    by = (sig(p[1:2]) + cgy_ref[...]) / gf
    w = jnp.exp(p[2:3]) * aw_ref[...]
    h = jnp.exp(p[3:4]) * ah_ref[...]
    x1 = bx - w / 2.0
    y1 = by - h / 2.0
    x2 = bx + w / 2.0
    y2 = by + h / 2.0
    area = (x2 - x1) * (y2 - y1)
    cl = p[5:_CH]
    m = jnp.max(cl, axis=0, keepdims=True)
    iot = lax.broadcasted_iota(jnp.int32, cl.shape, 0)
    am = jnp.min(jnp.where(cl == m, iot, 1000000), axis=0, keepdims=True)
    score = sig(p[4:5]) * sig(m)
    for ref, val in zip(out_ref, (x1, y1, x2, y2, area, score,
                                  am.astype(jnp.float32))):
        ref[...] = val[0]


_decode = pl.pallas_call(
    _decode_body,
    out_shape=[jax.ShapeDtypeStruct((_NPAD,), jnp.float32)] * 7,
)


def _nms_body(x1h, y1h, x2h, y2h, arh, sch, clh, out_hbm, x1v, y1v, x2v, y2v,
              arv, clv, effv, candv, tmpv, obufv, cshr):
    wid = lax.axis_index("s")
    base = wid * _SH
    iota = lax.iota(jnp.int32, 16)
    iotaf = iota.astype(jnp.float32)
    basef = (base * jnp.float32(1.0)).astype(jnp.float32)

    pltpu.sync_copy(x1h.at[pl.ds(base, _SH)], x1v)
    pltpu.sync_copy(y1h.at[pl.ds(base, _SH)], y1v)
    pltpu.sync_copy(x2h.at[pl.ds(base, _SH)], x2v)
    pltpu.sync_copy(y2h.at[pl.ds(base, _SH)], y2v)
    pltpu.sync_copy(arh.at[pl.ds(base, _SH)], arv)
    pltpu.sync_copy(sch.at[pl.ds(base, _SH)], effv)
    pltpu.sync_copy(clh.at[pl.ds(base, _SH)], clv)

    neg16 = jnp.full((16,), _NEG, jnp.float32)
    zero16 = jnp.zeros((16,), jnp.float32)

    def init_chunk(j, c):
        bv, bi = c
        sl = pl.ds(j * 16, 16)
        gidx = basef + (j * 16) * jnp.float32(1.0) + iotaf
        v = jnp.where(gidx >= jnp.float32(_N), jnp.float32(-1.0), effv[sl])
        effv[sl] = v
        upd = v > bv
        return jnp.where(upd, v, bv), jnp.where(upd, gidx, bi)

    bv0, bi0 = lax.fori_loop(0, _NCHUNK, init_chunk, (neg16, zero16))

    zi16 = jnp.zeros((16,), jnp.int32)
    b0x1 = plsc.load_gather(x1v, [zi16])
    b0y1 = plsc.load_gather(y1v, [zi16])
    b0x2 = plsc.load_gather(x2v, [zi16])
    b0y2 = plsc.load_gather(y2v, [zi16])
    b0cl = plsc.load_gather(clv, [zi16])

    def itbody(i, carry):
        bv, bi, nv = carry
        lm = jnp.max(bv)
        li = -jnp.max(jnp.where(bv == lm, -bi, jnp.float32(_NEG)))
        loff = (jnp.full((16,), li) - basef).astype(jnp.int32)
        row = jnp.where(iota == 0, jnp.full((16,), lm), zero16)
        row = jnp.where(iota == 1, jnp.full((16,), li), row)
        row = jnp.where(iota == 2, plsc.load_gather(x1v, [loff]), row)
        row = jnp.where(iota == 3, plsc.load_gather(y1v, [loff]), row)
        row = jnp.where(iota == 4, plsc.load_gather(x2v, [loff]), row)
        row = jnp.where(iota == 5, plsc.load_gather(y2v, [loff]), row)
        row = jnp.where(iota == 6, plsc.load_gather(arv, [loff]), row)
        row = jnp.where(iota == 7, plsc.load_gather(clv, [loff]), row)
        tmpv[...] = row
        pltpu.sync_copy(tmpv, cshr.at[wid, pl.ds(0, 16)])
        plsc.subcore_barrier()
        pltpu.sync_copy(cshr, candv)
        plsc.subcore_barrier()

        zi = jnp.zeros((16,), jnp.int32)
        s = plsc.load_gather(candv, [iota, zi])
        g = jnp.max(s)
        ownf = -jnp.max(jnp.where(s == g, -iotaf, jnp.float32(_NEG)))
        owni = jnp.full((16,), ownf).astype(jnp.int32)

        def fld(k):
            return plsc.load_gather(candv, [owni, jnp.full((16,), k, jnp.int32)])

        widxv = fld(1)
        wx1 = fld(2)
        wy1 = fld(3)
        wx2 = fld(4)
        wy2 = fld(5)
        wav = fld(6)
        wclv = fld(7)
        valid = g > jnp.float32(_SCORE_T)
        validv = jnp.full((16,), g) > jnp.float32(_SCORE_T)
        nv = nv + jnp.where(valid, jnp.float32(1.0), jnp.float32(0.0))

        @pl.when(wid == 0)
        def _():
            orow = jnp.where(iota == 0, jnp.where(validv, wx1, b0x1), zero16)
            orow = jnp.where(iota == 1, jnp.where(validv, wy1, b0y1), orow)
            orow = jnp.where(iota == 2, jnp.where(validv, wx2, b0x2), orow)
            orow = jnp.where(iota == 3, jnp.where(validv, wy2, b0y2), orow)
            orow = jnp.where(iota == 4,
                             jnp.where(validv, jnp.full((16,), g), zero16), orow)
            orow = jnp.where(iota == 5, jnp.where(validv, wclv, b0cl), orow)
            plsc.store_scatter(obufv, [jnp.full((16,), i, jnp.int32), iota], orow)

        owns = valid & (ownf == (wid * jnp.float32(1.0)))

        @pl.when(owns)
        def _():
            loffw = (widxv - basef).astype(jnp.int32)
            plsc.store_scatter(effv, [loffw], neg16, mask=iota == 0)

        def rw(j, c2):
            bv2, bi2 = c2
            sl = pl.ds(j * 16, 16)
            ix1 = jnp.maximum(wx1, x1v[sl])
            iy1 = jnp.maximum(wy1, y1v[sl])
            ix2 = jnp.minimum(wx2, x2v[sl])
            iy2 = jnp.minimum(wy2, y2v[sl])
            inter = jnp.maximum(ix2 - ix1, 0.0) * jnp.maximum(iy2 - iy1, 0.0)
            iou = inter / (wav + arv[sl] - inter + jnp.float32(1e-9))
            wgt = jnp.where(iou <= jnp.float32(_IOU_T),
                            jnp.exp(-iou * iou), jnp.float32(0.0))
            wgt = jnp.where(validv, wgt, jnp.float32(1.0))
            ce = effv[sl]
            ne = jnp.where(ce > jnp.float32(_GUARD), ce * wgt, ce)
            effv[sl] = ne
            gidx = basef + (j * 16) * jnp.float32(1.0) + iotaf
            upd = ne > bv2
            return jnp.where(upd, ne, bv2), jnp.where(upd, gidx, bi2)

        bv, bi = lax.fori_loop(0, _NCHUNK, rw, (neg16, zero16))
        return bv, bi, nv

    bv, bi, nv = lax.fori_loop(0, _MAXB, itbody, (bv0, bi0, jnp.float32(0.0)))

    @pl.when(wid == 0)
    def _():
        plsc.store_scatter(obufv, [jnp.full((16,), _MAXB, jnp.int32), iota],
                           jnp.full((16,), nv))
        pltpu.sync_copy(obufv, out_hbm)


@functools.cache
def _nms_call():
  return functools.partial(
    pl.kernel,
    out_type=jax.ShapeDtypeStruct((128, 16), jnp.float32),
    mesh=plsc.VectorSubcoreMesh(core_axis_name="c", subcore_axis_name="s",
                                num_cores=1, num_subcores=_NSUB),
    compiler_params=pltpu.CompilerParams(needs_layout_passes=False),
    scratch_types=[
        pltpu.VMEM((_SH,), jnp.float32),
        pltpu.VMEM((_SH,), jnp.float32),
        pltpu.VMEM((_SH,), jnp.float32),
        pltpu.VMEM((_SH,), jnp.float32),
        pltpu.VMEM((_SH,), jnp.float32),
        pltpu.VMEM((_SH,), jnp.float32),
        pltpu.VMEM((_SH,), jnp.float32),
        pltpu.VMEM((16, 128), jnp.float32),
        pltpu.VMEM((16,), jnp.float32),
        pltpu.VMEM((128, 16), jnp.float32),
        pltpu.VMEM_SHARED((16, 128), jnp.float32),
    ],
  )(_nms_body)


def kernel(pred0, pred1, pred2, anchors0, anchors1, anchors2):
    p = jnp.concatenate([
        pred0.reshape(-1, _CH),
        pred1.reshape(-1, _CH),
        pred2.reshape(-1, _CH),
        jnp.zeros((_NPAD - _N, _CH), jnp.float32),
    ], axis=0)
    pT = p.T

    def expand(anc, g):
        return jnp.tile(anc, (g * g, 1))

    anc = jnp.concatenate([expand(anchors0, 76), expand(anchors1, 38),
                           expand(anchors2, 19)], axis=0)
    aw = jnp.pad(anc[:, 0], (0, _NPAD - _N))[None]
    ah = jnp.pad(anc[:, 1], (0, _NPAD - _N))[None]

    dec = _decode(pT, jnp.asarray(_CGX), jnp.asarray(_CGY), jnp.asarray(_GF),
                  aw, ah)
    out = _nms_call()(*dec)
    boxes = out[:_MAXB, 0:4][None]
    scores = out[:_MAXB, 4][None]
    classes = out[:_MAXB, 5].astype(jnp.int32)[None]
    valid = out[_MAXB, 0].astype(jnp.int32)[None]
    return boxes, scores, classes, valid
